# baseline (speedup 1.0000x reference)
import functools

import jax
import jax.numpy as jnp
from jax import lax
from jax.experimental import pallas as pl
from jax.experimental.pallas import tpu as pltpu
from jax.experimental.pallas import tpu_sc as plsc

N_ROWS = 1000000
FEATS = 16
HFEATS = 8
BATCH_C = 16384
SLAB = 128

_info = plsc.get_sparse_core_info()
NC = _info.num_cores
NS = _info.num_subcores
LANES = _info.num_lanes
NW = NC * NS
B_PER_W = BATCH_C // NW
CHUNK = 128
N_BIAS_CHUNKS = B_PER_W // CHUNK
GROUPS = B_PER_W // LANES


def _mf_kernel(user_hbm, item_hbm, ub_hbm, ib_hbm, ue_hbm, ie_hbm, out_hbm,
               uidx_v, iidx_v, ubufA, ibufA, ubufB, ibufB,
               ub_v, ib_v, out_v, semA, semB, bsem):
    wid = lax.axis_index("s") * NC + lax.axis_index("c")
    base = wid * B_PER_W

    pltpu.sync_copy(user_hbm.at[pl.ds(base, B_PER_W)], uidx_v)
    pltpu.sync_copy(item_hbm.at[pl.ds(base, B_PER_W)], iidx_v)

    bias_copies = []
    for c in range(N_BIAS_CHUNKS):
        s = pl.ds(c * CHUNK, CHUNK)
        bias_copies.append(pltpu.make_async_copy(ub_hbm.at[uidx_v.at[s]], ub_v.at[s], bsem))
        bias_copies.append(pltpu.make_async_copy(ib_hbm.at[iidx_v.at[s]], ib_v.at[s], bsem))
    for cp in bias_copies:
        cp.start()

    lane_iota = lax.broadcasted_iota(jnp.int32, (LANES,), 0)

    def issue_stage(g, h, ubuf, ibuf, sem):
        e0 = g * LANES
        u16 = uidx_v[pl.ds(e0, LANES)]
        i16 = iidx_v[pl.ds(e0, LANES)]
        for l in range(LANES):
            cu = pl.multiple_of((u16[l] // SLAB) * SLAB, SLAB)
            ci = pl.multiple_of((i16[l] // SLAB) * SLAB, SLAB)
            pltpu.make_async_copy(
                ue_hbm.at[pl.ds(h * HFEATS, HFEATS), pl.ds(cu, SLAB)],
                ubuf.at[l], sem).start()
            pltpu.make_async_copy(
                ie_hbm.at[pl.ds(h * HFEATS, HFEATS), pl.ds(ci, SLAB)],
                ibuf.at[l], sem).start()

    def drain_stage(ubuf, ibuf, sem):
        for l in range(LANES):
            pltpu.make_async_copy(
                ue_hbm.at[pl.ds(0, HFEATS), pl.ds(0, SLAB)], ubuf.at[l], sem
            ).wait()
            pltpu.make_async_copy(
                ie_hbm.at[pl.ds(0, HFEATS), pl.ds(0, SLAB)], ibuf.at[l], sem
            ).wait()

    def compute_stage(g, ubuf, ibuf, acc):
        e0 = g * LANES
        cu16 = lax.rem(uidx_v[pl.ds(e0, LANES)], SLAB)
        ci16 = lax.rem(iidx_v[pl.ds(e0, LANES)], SLAB)
        for f in range(HFEATS):
            fvec = jnp.full((LANES,), f, jnp.int32)
            vu = plsc.load_gather(ubuf, [lane_iota, fvec, cu16])
            vi = plsc.load_gather(ibuf, [lane_iota, fvec, ci16])
            acc = acc + vu * vi
        return acc

    issue_stage(0, 0, ubufA, ibufA, semA)
    for cp in bias_copies:
        cp.wait()

    def group_body(g, _):
        e0 = g * LANES
        issue_stage(g, 1, ubufB, ibufB, semB)
        drain_stage(ubufA, ibufA, semA)
        acc = ub_v[pl.ds(e0, LANES)] + ib_v[pl.ds(e0, LANES)]
        acc = compute_stage(g, ubufA, ibufA, acc)

        @pl.when(g < GROUPS - 1)
        def _():
            issue_stage(g + 1, 0, ubufA, ibufA, semA)

        drain_stage(ubufB, ibufB, semB)
        acc = compute_stage(g, ubufB, ibufB, acc)
        out_v[pl.ds(e0, LANES)] = acc
        return 0

    lax.fori_loop(0, GROUPS, group_body, 0)

    pltpu.sync_copy(out_v, out_hbm.at[pl.ds(base, B_PER_W)])


@jax.jit
def _mf(user, item, u_bias_flat, i_bias_flat, ue_t, ie_t):
    mesh = plsc.VectorSubcoreMesh(core_axis_name="c", subcore_axis_name="s")
    return pl.kernel(
        _mf_kernel,
        out_type=jax.ShapeDtypeStruct((BATCH_C,), jnp.float32),
        mesh=mesh,
        compiler_params=pltpu.CompilerParams(needs_layout_passes=False, use_tc_tiling_on_sc=True),
        scratch_types=[
            pltpu.VMEM((B_PER_W,), jnp.int32),
            pltpu.VMEM((B_PER_W,), jnp.int32),
            pltpu.VMEM((LANES, HFEATS, SLAB), jnp.float32),
            pltpu.VMEM((LANES, HFEATS, SLAB), jnp.float32),
            pltpu.VMEM((LANES, HFEATS, SLAB), jnp.float32),
            pltpu.VMEM((LANES, HFEATS, SLAB), jnp.float32),
            pltpu.VMEM((B_PER_W,), jnp.float32),
            pltpu.VMEM((B_PER_W,), jnp.float32),
            pltpu.VMEM((B_PER_W,), jnp.float32),
            pltpu.SemaphoreType.DMA,
            pltpu.SemaphoreType.DMA,
            pltpu.SemaphoreType.DMA,
        ],
    )(user, item, u_bias_flat, i_bias_flat, ue_t, ie_t)


def kernel(user, item, u_bias, i_bias, u_embed, i_embed):
    return _mf(
        user.astype(jnp.int32),
        item.astype(jnp.int32),
        u_bias.reshape(-1),
        i_bias.reshape(-1),
        u_embed.T,
        i_embed.T,
    )
